# Initial kernel scaffold; baseline (speedup 1.0000x reference)
import jax
import jax.numpy as jnp
from jax.experimental import pallas as pl


def kernel(x, edge_index, edge_weight, edge_indexZ, edge_weightZ, M, Z, W0, b0, gamma0, beta0, W1, b1, gamma1, beta1, W2, b2):
    raise NotImplementedError("write your pallas kernel here")


# speedup vs baseline: 9.8024x; 9.8024x over previous
import functools

import jax
import jax.numpy as jnp
from jax import lax
from jax.experimental import pallas as pl
from jax.experimental.pallas import tpu as pltpu
from jax.experimental.pallas import tpu_sc as plsc

N = 10000
E = 320000
F_IN = 128
H = 16
C = 40
INV_BN = 1.0 / (1.0 + 1e-5) ** 0.5

NC = 2
NS = 16
L = 16
NW = NC * NS
EPT = E // NW
K = 80
NCH = EPT // K
ROWS_T = 624
TAIL = N - ROWS_T * NS

_GDN = lax.GatherDimensionNumbers(
    offset_dims=(), collapsed_slice_dims=(0,), start_index_map=(0,))


def _lane_bcast(v, r):
    idx = jnp.full((L, 1), r, dtype=jnp.int32)
    return lax.gather(v, idx, _GDN, (1,),
                      mode=lax.GatherScatterMode.PROMISE_IN_BOUNDS)


def _spmm_sc(y, src3, dst4, w3, zeros):
    mesh = plsc.VectorSubcoreMesh(core_axis_name="c", subcore_axis_name="s")

    @functools.partial(
        pl.kernel,
        mesh=mesh,
        out_type=jax.ShapeDtypeStruct((NC, N, H), jnp.float32),
        scratch_types=[
            pltpu.VMEM((EPT,), jnp.int32),
            pltpu.VMEM((NCH, K), jnp.int32),
            pltpu.VMEM((EPT,), jnp.float32),
            pltpu.VMEM((K, L), jnp.float32),
            pltpu.VMEM_SHARED((N, H), jnp.float32),
            pltpu.SemaphoreType.DMA,
        ],
        compiler_params=pltpu.CompilerParams(use_tc_tiling_on_sc=False),
    )
    def k(y_hbm, src_hbm, dst_hbm, w_hbm, z_hbm, out_hbm,
          src_v, dst_v, w_v, rows_v, acc, sem):
        c = lax.axis_index("c")
        s = lax.axis_index("s")
        pltpu.sync_copy(src_hbm.at[c, s], src_v)
        pltpu.sync_copy(dst_hbm.at[c, s], dst_v)
        pltpu.sync_copy(w_hbm.at[c, s], w_v)
        rbase = s * ROWS_T
        pltpu.sync_copy(z_hbm.at[pl.ds(rbase, ROWS_T)],
                        acc.at[pl.ds(rbase, ROWS_T)])

        @pl.when(s == 0)
        def _zero_tail():
            pltpu.sync_copy(z_hbm.at[pl.ds(ROWS_T * NS, TAIL)],
                            acc.at[pl.ds(ROWS_T * NS, TAIL)])

        plsc.subcore_barrier()

        def chunk(j, carry):
            ebase = j * K
            pltpu.async_copy(y_hbm.at[src_v.at[pl.ds(ebase, K)]],
                             rows_v, sem).wait()
            for jj in range(K // L):
                w16 = w_v[pl.ds(ebase + jj * L, L)]
                for r in range(L):
                    e = jj * L + r
                    rows_v[e] = rows_v[e] * _lane_bcast(w16, r)
            pltpu.sync_copy(rows_v, acc.at[dst_v.at[j]], add=True)
            return carry

        lax.fori_loop(0, NCH, chunk, 0)
        plsc.subcore_barrier()
        pltpu.sync_copy(acc.at[pl.ds(rbase, ROWS_T)],
                        out_hbm.at[c, pl.ds(rbase, ROWS_T)])

        @pl.when(s == 0)
        def _write_tail():
            pltpu.sync_copy(acc.at[pl.ds(ROWS_T * NS, TAIL)],
                            out_hbm.at[c, pl.ds(ROWS_T * NS, TAIL)])

    return k(y, src3, dst4, w3, zeros)


def _dense_in(x, M, W0):
    def body(x_ref, m_ref, w_ref, o_ref):
        o_ref[...] = jnp.dot(x_ref[...] * m_ref[...], w_ref[...],
                             preferred_element_type=jnp.float32)
    return pl.pallas_call(
        body, out_shape=jax.ShapeDtypeStruct((N, H), jnp.float32))(x, M, W0)


def _post0(sa, sb, Z, b0, g0, be0):
    def body(a_ref, b_ref, z_ref, bias_ref, g_ref, be_ref, o_ref):
        t = (a_ref[...] + b_ref[...]) * z_ref[...] + bias_ref[...]
        o_ref[...] = jnp.maximum(g_ref[...] * t * INV_BN + be_ref[...], 0.0)
    return pl.pallas_call(
        body, out_shape=jax.ShapeDtypeStruct((N, H), jnp.float32))(
            sa, sb, Z, b0, g0, be0)


def _post1(sa, sb, W1, b1, g1, be1):
    def body(a_ref, b_ref, w_ref, bias_ref, g_ref, be_ref, o_ref):
        t = jnp.dot(a_ref[...] + b_ref[...], w_ref[...],
                    preferred_element_type=jnp.float32) + bias_ref[...]
        o_ref[...] = jnp.maximum(g_ref[...] * t * INV_BN + be_ref[...], 0.0)
    return pl.pallas_call(
        body, out_shape=jax.ShapeDtypeStruct((N, H), jnp.float32))(
            sa, sb, W1, b1, g1, be1)


def _post2(sa, sb, W2, b2):
    def body(a_ref, b_ref, w_ref, bias_ref, o_ref):
        t = jnp.dot(a_ref[...] + b_ref[...], w_ref[...],
                    preferred_element_type=jnp.float32) + bias_ref[...]
        m = jnp.max(t, axis=-1, keepdims=True)
        lse = jnp.log(jnp.sum(jnp.exp(t - m), axis=-1, keepdims=True)) + m
        o_ref[...] = t - lse
    return pl.pallas_call(
        body, out_shape=jax.ShapeDtypeStruct((N, C), jnp.float32))(
            sa, sb, W2, b2)


def _edge_prep(edge_index, edge_weight):
    src = edge_index[0].astype(jnp.int32).reshape(NC, NS, EPT)
    dst = edge_index[1].astype(jnp.int32).reshape(NC, NS, NCH, K)
    w = edge_weight.reshape(NC, NS, EPT)
    return src, dst, w


def kernel(x, edge_index, edge_weight, edge_indexZ, edge_weightZ, M, Z,
           W0, b0, gamma0, beta0, W1, b1, gamma1, beta1, W2, b2):
    srcZ, dstZ, wZ = _edge_prep(edge_indexZ, edge_weightZ)
    src, dst, w = _edge_prep(edge_index, edge_weight)
    zeros = jnp.zeros((N, H), jnp.float32)

    y0 = _dense_in(x, M, W0)
    s0 = _spmm_sc(y0, srcZ, dstZ, wZ, zeros)
    h0 = _post0(s0[0], s0[1], Z, b0.reshape(1, H),
                gamma0.reshape(1, H), beta0.reshape(1, H))
    s1 = _spmm_sc(h0, src, dst, w, zeros)
    h1 = _post1(s1[0], s1[1], W1, b1.reshape(1, H),
                gamma1.reshape(1, H), beta1.reshape(1, H))
    s2 = _spmm_sc(h1, src, dst, w, zeros)
    return _post2(s2[0], s2[1], W2, b2.reshape(1, C))


# speedup vs baseline: 20.7863x; 2.1205x over previous
import functools

import jax
import jax.numpy as jnp
from jax import lax
from jax.experimental import pallas as pl
from jax.experimental.pallas import tpu as pltpu
from jax.experimental.pallas import tpu_sc as plsc

N = 10000
E = 320000
F_IN = 128
H = 16
C = 40
INV_BN = 1.0 / (1.0 + 1e-5) ** 0.5

NC = 2
NS = 16
L = 16
NW = NC * NS
EPT = E // NW
K = 80
NCH = EPT // K
NBUF = 5
RND = NCH // NBUF
ROWS_T = 624
TAIL = N - ROWS_T * NS

_GDN = lax.GatherDimensionNumbers(
    offset_dims=(), collapsed_slice_dims=(0,), start_index_map=(0,))


def _lane_bcast(v, r):
    idx = jnp.full((L, 1), r, dtype=jnp.int32)
    return lax.gather(v, idx, _GDN, (1,),
                      mode=lax.GatherScatterMode.PROMISE_IN_BOUNDS)


def _spmm_sc(y, src3, dst4, w3, zeros):
    mesh = plsc.VectorSubcoreMesh(core_axis_name="c", subcore_axis_name="s")

    @functools.partial(
        pl.kernel,
        mesh=mesh,
        out_type=jax.ShapeDtypeStruct((NC, N, H), jnp.float32),
        scratch_types=(
            [pltpu.VMEM((EPT,), jnp.int32),
             pltpu.VMEM((NCH, K), jnp.int32),
             pltpu.VMEM((EPT,), jnp.float32),
             pltpu.VMEM_SHARED((N, H), jnp.float32)]
            + [pltpu.VMEM((K, L), jnp.float32)] * (2 * NBUF)
            + [pltpu.SemaphoreType.DMA] * (2 * NBUF)
        ),
        compiler_params=pltpu.CompilerParams(use_tc_tiling_on_sc=False),
    )
    def k(y_hbm, src_hbm, dst_hbm, w_hbm, z_hbm, out_hbm,
          src_v, dst_v, w_v, acc, *scr):
        gbuf = scr[0:NBUF]
        sbuf = scr[NBUF:2 * NBUF]
        gsem = scr[2 * NBUF:3 * NBUF]
        ssem = scr[3 * NBUF:4 * NBUF]
        c = lax.axis_index("c")
        s = lax.axis_index("s")
        pltpu.sync_copy(src_hbm.at[c, s], src_v)
        pltpu.sync_copy(dst_hbm.at[c, s], dst_v)
        pltpu.sync_copy(w_hbm.at[c, s], w_v)
        rbase = s * ROWS_T
        pltpu.sync_copy(z_hbm.at[pl.ds(rbase, ROWS_T)],
                        acc.at[pl.ds(rbase, ROWS_T)])

        @pl.when(s == 0)
        def _zero_tail():
            pltpu.sync_copy(z_hbm.at[pl.ds(ROWS_T * NS, TAIL)],
                            acc.at[pl.ds(ROWS_T * NS, TAIL)])

        plsc.subcore_barrier()

        def g_start(jj, b):
            pltpu.async_copy(y_hbm.at[src_v.at[pl.ds(jj * K, K)]],
                             gbuf[b], gsem[b])

        def g_wait(b):
            pltpu.make_async_copy(y_hbm.at[src_v.at[pl.ds(0, K)]],
                                  gbuf[b], gsem[b]).wait()

        def s_start(jj, b):
            pltpu.async_copy(sbuf[b], acc.at[dst_v.at[jj]], ssem[b], add=True)

        def s_wait(b):
            pltpu.make_async_copy(sbuf[b], acc.at[dst_v.at[0]], ssem[b]).wait()

        def compute(jj, b):
            for q in range(K // L):
                w16 = w_v[pl.ds(jj * K + q * L, L)]
                for r in range(L):
                    e = q * L + r
                    sbuf[b][e] = gbuf[b][e] * _lane_bcast(w16, r)

        for b in range(NBUF):
            g_start(b, b)

        def round_body(rr, carry):
            for b in range(NBUF):
                jj = rr * NBUF + b
                g_wait(b)

                @pl.when(rr > 0)
                def _drain():
                    s_wait(b)

                compute(jj, b)
                s_start(jj, b)

                @pl.when(rr < RND - 1)
                def _prefetch():
                    g_start(jj + NBUF, b)

            return carry

        lax.fori_loop(0, RND, round_body, 0)
        for b in range(NBUF):
            s_wait(b)
        plsc.subcore_barrier()
        pltpu.sync_copy(acc.at[pl.ds(rbase, ROWS_T)],
                        out_hbm.at[c, pl.ds(rbase, ROWS_T)])

        @pl.when(s == 0)
        def _write_tail():
            pltpu.sync_copy(acc.at[pl.ds(ROWS_T * NS, TAIL)],
                            out_hbm.at[c, pl.ds(ROWS_T * NS, TAIL)])

    return k(y, src3, dst4, w3, zeros)


def _dense_in(x, M, W0):
    def body(x_ref, m_ref, w_ref, o_ref):
        o_ref[...] = jnp.dot(x_ref[...] * m_ref[...], w_ref[...],
                             preferred_element_type=jnp.float32)
    return pl.pallas_call(
        body, out_shape=jax.ShapeDtypeStruct((N, H), jnp.float32))(x, M, W0)


def _post0(sa, sb, Z, b0, g0, be0):
    def body(a_ref, b_ref, z_ref, bias_ref, g_ref, be_ref, o_ref):
        t = (a_ref[...] + b_ref[...]) * z_ref[...] + bias_ref[...]
        o_ref[...] = jnp.maximum(g_ref[...] * t * INV_BN + be_ref[...], 0.0)
    return pl.pallas_call(
        body, out_shape=jax.ShapeDtypeStruct((N, H), jnp.float32))(
            sa, sb, Z, b0, g0, be0)


def _post1(sa, sb, W1, b1, g1, be1):
    def body(a_ref, b_ref, w_ref, bias_ref, g_ref, be_ref, o_ref):
        t = jnp.dot(a_ref[...] + b_ref[...], w_ref[...],
                    preferred_element_type=jnp.float32) + bias_ref[...]
        o_ref[...] = jnp.maximum(g_ref[...] * t * INV_BN + be_ref[...], 0.0)
    return pl.pallas_call(
        body, out_shape=jax.ShapeDtypeStruct((N, H), jnp.float32))(
            sa, sb, W1, b1, g1, be1)


def _post2(sa, sb, W2, b2):
    def body(a_ref, b_ref, w_ref, bias_ref, o_ref):
        t = jnp.dot(a_ref[...] + b_ref[...], w_ref[...],
                    preferred_element_type=jnp.float32) + bias_ref[...]
        m = jnp.max(t, axis=-1, keepdims=True)
        lse = jnp.log(jnp.sum(jnp.exp(t - m), axis=-1, keepdims=True)) + m
        o_ref[...] = t - lse
    return pl.pallas_call(
        body, out_shape=jax.ShapeDtypeStruct((N, C), jnp.float32))(
            sa, sb, W2, b2)


def _edge_prep(edge_index, edge_weight):
    src = edge_index[0].astype(jnp.int32).reshape(NC, NS, EPT)
    dst = edge_index[1].astype(jnp.int32).reshape(NC, NS, NCH, K)
    w = edge_weight.reshape(NC, NS, EPT)
    return src, dst, w


def kernel(x, edge_index, edge_weight, edge_indexZ, edge_weightZ, M, Z,
           W0, b0, gamma0, beta0, W1, b1, gamma1, beta1, W2, b2):
    srcZ, dstZ, wZ = _edge_prep(edge_indexZ, edge_weightZ)
    src, dst, w = _edge_prep(edge_index, edge_weight)
    zeros = jnp.zeros((N, H), jnp.float32)

    y0 = _dense_in(x, M, W0)
    s0 = _spmm_sc(y0, srcZ, dstZ, wZ, zeros)
    h0 = _post0(s0[0], s0[1], Z, b0.reshape(1, H),
                gamma0.reshape(1, H), beta0.reshape(1, H))
    s1 = _spmm_sc(h0, src, dst, w, zeros)
    h1 = _post1(s1[0], s1[1], W1, b1.reshape(1, H),
                gamma1.reshape(1, H), beta1.reshape(1, H))
    s2 = _spmm_sc(h1, src, dst, w, zeros)
    return _post2(s2[0], s2[1], W2, b2.reshape(1, C))
